# Initial kernel scaffold; baseline (speedup 1.0000x reference)
import functools

import jax
import jax.numpy as jnp
from jax import lax
from jax.experimental import pallas as pl
from jax.experimental.pallas import tpu as pltpu
from jax.experimental.pallas import tpu_sc as plsc

N = 10000
E = 320000
D = 128
H = 128

NC = 2
NS = 16
NW = NC * NS
E_PER_TILE = E // NW
CHUNK = 80
N_CHUNKS = E_PER_TILE // CHUNK
ROWS_PER_TILE = N // NS
ZROWS = 125


def _sc_body(nf_hbm, src_hbm, dst_hbm, out_hbm,
             acc, src_v, dst_v, rows_v, zbuf, sem):
    cid = lax.axis_index("c")
    sid = lax.axis_index("s")
    wid = sid * NC + cid

    zvec = jnp.zeros((16,), jnp.float32)

    def zero_body(i, carry):
        r = i // (D // 16)
        col = (i % (D // 16)) * 16
        zbuf[r, pl.ds(col, 16)] = zvec
        return carry

    lax.fori_loop(0, ZROWS * (D // 16), zero_body, 0)

    row0 = sid * ROWS_PER_TILE

    def zcopy_body(j, carry):
        pltpu.sync_copy(zbuf, acc.at[pl.ds(row0 + j * ZROWS, ZROWS)])
        return carry

    lax.fori_loop(0, ROWS_PER_TILE // ZROWS, zcopy_body, 0)
    plsc.subcore_barrier()

    ebase = wid * E_PER_TILE

    def chunk_body(i, carry):
        base = ebase + i * CHUNK
        pltpu.sync_copy(src_hbm.at[pl.ds(base, CHUNK)], src_v)
        pltpu.sync_copy(dst_hbm.at[pl.ds(base, CHUNK)], dst_v)
        pltpu.async_copy(nf_hbm.at[src_v], rows_v, sem).wait()
        pltpu.sync_copy(rows_v, acc.at[dst_v], add=True)
        return carry

    lax.fori_loop(0, N_CHUNKS, chunk_body, 0)
    plsc.subcore_barrier()

    pltpu.sync_copy(acc.at[pl.ds(row0, ROWS_PER_TILE)],
                    out_hbm.at[cid, pl.ds(row0, ROWS_PER_TILE)])


def _sc_aggregate(node_feature, src, dst):
    mesh = plsc.VectorSubcoreMesh(core_axis_name="c", subcore_axis_name="s")
    k = pl.kernel(
        _sc_body,
        out_type=jax.ShapeDtypeStruct((NC, N, D), jnp.float32),
        mesh=mesh,
        scratch_types=[
            pltpu.VMEM_SHARED((N, D), jnp.float32),
            pltpu.VMEM((CHUNK,), jnp.int32),
            pltpu.VMEM((CHUNK,), jnp.int32),
            pltpu.VMEM((CHUNK, D), jnp.float32),
            pltpu.VMEM((ZROWS, D), jnp.float32),
            pltpu.SemaphoreType.DMA,
        ],
    )
    return k(node_feature, src, dst)


def _mlp_body(parts_ref, nf_ref, w1a_ref, w1b_ref, b1_ref,
              w2_ref, b2_ref, w3_ref, b3_ref, out_ref):
    agg = parts_ref[0] + parts_ref[1]
    h = jnp.dot(agg, w1a_ref[...], preferred_element_type=jnp.float32)
    h += jnp.dot(nf_ref[...], w1b_ref[...], preferred_element_type=jnp.float32)
    h = jnp.maximum(h + b1_ref[...], 0.0)
    h = jnp.dot(h, w2_ref[...], preferred_element_type=jnp.float32)
    h = jnp.maximum(h + b2_ref[...], 0.0)
    h = jnp.dot(h, w3_ref[...], preferred_element_type=jnp.float32)
    out_ref[...] = h + b3_ref[...]


def _mlp(parts, node_feature, W1, b1, W2, b2, W3, b3):
    R = 1000
    grid = (N // R,)
    w1a = W1[:D]
    w1b = W1[D:]
    return pl.pallas_call(
        _mlp_body,
        grid=grid,
        in_specs=[
            pl.BlockSpec((NC, R, D), lambda i: (0, i, 0)),
            pl.BlockSpec((R, D), lambda i: (i, 0)),
            pl.BlockSpec((D, H), lambda i: (0, 0)),
            pl.BlockSpec((D, H), lambda i: (0, 0)),
            pl.BlockSpec((1, H), lambda i: (0, 0)),
            pl.BlockSpec((H, H), lambda i: (0, 0)),
            pl.BlockSpec((1, H), lambda i: (0, 0)),
            pl.BlockSpec((H, D), lambda i: (0, 0)),
            pl.BlockSpec((1, D), lambda i: (0, 0)),
        ],
        out_specs=pl.BlockSpec((R, D), lambda i: (i, 0)),
        out_shape=jax.ShapeDtypeStruct((N, D), jnp.float32),
    )(parts, node_feature, w1a, w1b, b1.reshape(1, H),
      W2, b2.reshape(1, H), W3, b3.reshape(1, D))


@jax.jit
def kernel(node_feature, edge_index, W1, b1, W2, b2, W3, b3):
    src = edge_index[0]
    dst = edge_index[1]
    parts = _sc_aggregate(node_feature, src, dst)
    return _mlp(parts, node_feature, W1, b1, W2, b2, W3, b3)


# speedup vs baseline: 5.4961x; 5.4961x over previous
import functools

import jax
import jax.numpy as jnp
from jax import lax
from jax.experimental import pallas as pl
from jax.experimental.pallas import tpu as pltpu
from jax.experimental.pallas import tpu_sc as plsc

N = 10000
E = 320000
D = 128
H = 128

NC = 2
NS = 16
NW = NC * NS
E_PER_TILE = E // NW
CHUNK = 80
N_CHUNKS = E_PER_TILE // CHUNK
ROWS_MAIN = 624
ROWS_TAIL = N - (NS - 1) * ROWS_MAIN
ZROWS = 16


def _sc_body(nf_hbm, src_hbm, dst_hbm, out_hbm,
             acc, src_v, dst_v, rows_v, zbuf, sem):
    cid = lax.axis_index("c")
    sid = lax.axis_index("s")
    wid = sid * NC + cid

    zvec = jnp.zeros((16,), jnp.float32)

    def zero_body(i, carry):
        r = i // (D // 16)
        col = (i % (D // 16)) * 16
        zbuf[r, pl.ds(col, 16)] = zvec
        return carry

    lax.fori_loop(0, ZROWS * (D // 16), zero_body, 0)

    row0 = sid * ROWS_MAIN

    def zcopy_body(j, carry):
        pltpu.sync_copy(zbuf, acc.at[pl.ds(row0 + j * ZROWS, ZROWS)])
        return carry

    lax.fori_loop(0, ROWS_MAIN // ZROWS, zcopy_body, 0)

    @pl.when(sid == NS - 1)
    def _zero_tail():
        pltpu.sync_copy(zbuf, acc.at[pl.ds(N - ZROWS, ZROWS)])

    plsc.subcore_barrier()

    ebase = wid * E_PER_TILE

    def chunk_body(i, carry):
        base = ebase + i * CHUNK
        pltpu.sync_copy(src_hbm.at[pl.ds(base, CHUNK)], src_v)
        pltpu.sync_copy(dst_hbm.at[pl.ds(base, CHUNK)], dst_v)
        pltpu.async_copy(nf_hbm.at[src_v], rows_v, sem).wait()
        pltpu.sync_copy(rows_v, acc.at[dst_v], add=True)
        return carry

    lax.fori_loop(0, N_CHUNKS, chunk_body, 0)
    plsc.subcore_barrier()

    pltpu.sync_copy(acc.at[pl.ds(row0, ROWS_MAIN)],
                    out_hbm.at[cid, pl.ds(row0, ROWS_MAIN)])

    @pl.when(sid == NS - 1)
    def _write_tail():
        pltpu.sync_copy(acc.at[pl.ds(N - ZROWS, ZROWS)],
                        out_hbm.at[cid, pl.ds(N - ZROWS, ZROWS)])


def _sc_aggregate(node_feature, src, dst):
    mesh = plsc.VectorSubcoreMesh(core_axis_name="c", subcore_axis_name="s")
    k = pl.kernel(
        _sc_body,
        out_type=jax.ShapeDtypeStruct((NC, N, D), jnp.float32),
        mesh=mesh,
        scratch_types=[
            pltpu.VMEM_SHARED((N, D), jnp.float32),
            pltpu.VMEM((CHUNK,), jnp.int32),
            pltpu.VMEM((CHUNK,), jnp.int32),
            pltpu.VMEM((CHUNK, D), jnp.float32),
            pltpu.VMEM((ZROWS, D), jnp.float32),
            pltpu.SemaphoreType.DMA,
        ],
    )
    return k(node_feature, src, dst)


def _mlp_body(parts_ref, nf_ref, w1a_ref, w1b_ref, b1_ref,
              w2_ref, b2_ref, w3_ref, b3_ref, out_ref):
    agg = parts_ref[0] + parts_ref[1]
    h = jnp.dot(agg, w1a_ref[...], preferred_element_type=jnp.float32)
    h += jnp.dot(nf_ref[...], w1b_ref[...], preferred_element_type=jnp.float32)
    h = jnp.maximum(h + b1_ref[...], 0.0)
    h = jnp.dot(h, w2_ref[...], preferred_element_type=jnp.float32)
    h = jnp.maximum(h + b2_ref[...], 0.0)
    h = jnp.dot(h, w3_ref[...], preferred_element_type=jnp.float32)
    out_ref[...] = h + b3_ref[...]


def _mlp(parts, node_feature, W1, b1, W2, b2, W3, b3):
    R = 1000
    grid = (N // R,)
    w1a = W1[:D]
    w1b = W1[D:]
    return pl.pallas_call(
        _mlp_body,
        grid=grid,
        in_specs=[
            pl.BlockSpec((NC, R, D), lambda i: (0, i, 0)),
            pl.BlockSpec((R, D), lambda i: (i, 0)),
            pl.BlockSpec((D, H), lambda i: (0, 0)),
            pl.BlockSpec((D, H), lambda i: (0, 0)),
            pl.BlockSpec((1, H), lambda i: (0, 0)),
            pl.BlockSpec((H, H), lambda i: (0, 0)),
            pl.BlockSpec((1, H), lambda i: (0, 0)),
            pl.BlockSpec((H, D), lambda i: (0, 0)),
            pl.BlockSpec((1, D), lambda i: (0, 0)),
        ],
        out_specs=pl.BlockSpec((R, D), lambda i: (i, 0)),
        out_shape=jax.ShapeDtypeStruct((N, D), jnp.float32),
    )(parts, node_feature, w1a, w1b, b1.reshape(1, H),
      W2, b2.reshape(1, H), W3, b3.reshape(1, D))


@jax.jit
def kernel(node_feature, edge_index, W1, b1, W2, b2, W3, b3):
    src = edge_index[0]
    dst = edge_index[1]
    parts = _sc_aggregate(node_feature, src, dst)
    return _mlp(parts, node_feature, W1, b1, W2, b2, W3, b3)


# speedup vs baseline: 7.6053x; 1.3838x over previous
import functools

import jax
import jax.numpy as jnp
from jax import lax
from jax.experimental import pallas as pl
from jax.experimental.pallas import tpu as pltpu
from jax.experimental.pallas import tpu_sc as plsc

N = 10000
E = 320000
D = 128
H = 128

NC = 2
NS = 16
NW = NC * NS
E_PER_TILE = E // NW
CHUNK = 80
N_CHUNKS = E_PER_TILE // CHUNK
ROWS_MAIN = 624
ROWS_TAIL = N - (NS - 1) * ROWS_MAIN
ZROWS = 16


def _sc_body(nf_hbm, src_hbm, dst_hbm, out_hbm,
             acc, src_v, dst_v, rows_v, zbuf, sem):
    cid = lax.axis_index("c")
    sid = lax.axis_index("s")
    wid = sid * NC + cid

    zvec = jnp.zeros((16,), jnp.float32)

    def zero_body(i, carry):
        r = i // (D // 16)
        col = (i % (D // 16)) * 16
        zbuf[r, pl.ds(col, 16)] = zvec
        return carry

    lax.fori_loop(0, ZROWS * (D // 16), zero_body, 0)

    row0 = sid * ROWS_MAIN

    def zcopy_body(j, carry):
        pltpu.sync_copy(zbuf, acc.at[pl.ds(row0 + j * ZROWS, ZROWS)])
        return carry

    lax.fori_loop(0, ROWS_MAIN // ZROWS, zcopy_body, 0)

    @pl.when(sid == NS - 1)
    def _zero_tail():
        pltpu.sync_copy(zbuf, acc.at[pl.ds(N - ZROWS, ZROWS)])

    plsc.subcore_barrier()

    pltpu.sync_copy(src_hbm.at[wid], src_v)
    pltpu.sync_copy(dst_hbm.at[wid], dst_v)

    def chunk_body(i, carry):
        pltpu.async_copy(nf_hbm.at[src_v.at[i]], rows_v, sem).wait()
        pltpu.sync_copy(rows_v, acc.at[dst_v.at[i]], add=True)
        return carry

    lax.fori_loop(0, N_CHUNKS, chunk_body, 0)
    plsc.subcore_barrier()

    pltpu.sync_copy(acc.at[pl.ds(row0, ROWS_MAIN)],
                    out_hbm.at[cid, pl.ds(row0, ROWS_MAIN)])

    @pl.when(sid == NS - 1)
    def _write_tail():
        pltpu.sync_copy(acc.at[pl.ds(N - ZROWS, ZROWS)],
                        out_hbm.at[cid, pl.ds(N - ZROWS, ZROWS)])


def _sc_aggregate(node_feature, src, dst):
    mesh = plsc.VectorSubcoreMesh(core_axis_name="c", subcore_axis_name="s")
    k = pl.kernel(
        _sc_body,
        out_type=jax.ShapeDtypeStruct((NC, N, D), jnp.float32),
        mesh=mesh,
        scratch_types=[
            pltpu.VMEM_SHARED((N, D), jnp.float32),
            pltpu.VMEM((N_CHUNKS, CHUNK), jnp.int32),
            pltpu.VMEM((N_CHUNKS, CHUNK), jnp.int32),
            pltpu.VMEM((CHUNK, D), jnp.float32),
            pltpu.VMEM((ZROWS, D), jnp.float32),
            pltpu.SemaphoreType.DMA,
        ],
    )
    return k(node_feature,
             src.reshape(NW, N_CHUNKS, CHUNK),
             dst.reshape(NW, N_CHUNKS, CHUNK))


def _mlp_body(parts_ref, nf_ref, w1a_ref, w1b_ref, b1_ref,
              w2_ref, b2_ref, w3_ref, b3_ref, out_ref):
    agg = parts_ref[0] + parts_ref[1]
    h = jnp.dot(agg, w1a_ref[...], preferred_element_type=jnp.float32)
    h += jnp.dot(nf_ref[...], w1b_ref[...], preferred_element_type=jnp.float32)
    h = jnp.maximum(h + b1_ref[...], 0.0)
    h = jnp.dot(h, w2_ref[...], preferred_element_type=jnp.float32)
    h = jnp.maximum(h + b2_ref[...], 0.0)
    h = jnp.dot(h, w3_ref[...], preferred_element_type=jnp.float32)
    out_ref[...] = h + b3_ref[...]


def _mlp(parts, node_feature, W1, b1, W2, b2, W3, b3):
    R = 1000
    grid = (N // R,)
    w1a = W1[:D]
    w1b = W1[D:]
    return pl.pallas_call(
        _mlp_body,
        grid=grid,
        in_specs=[
            pl.BlockSpec((NC, R, D), lambda i: (0, i, 0)),
            pl.BlockSpec((R, D), lambda i: (i, 0)),
            pl.BlockSpec((D, H), lambda i: (0, 0)),
            pl.BlockSpec((D, H), lambda i: (0, 0)),
            pl.BlockSpec((1, H), lambda i: (0, 0)),
            pl.BlockSpec((H, H), lambda i: (0, 0)),
            pl.BlockSpec((1, H), lambda i: (0, 0)),
            pl.BlockSpec((H, D), lambda i: (0, 0)),
            pl.BlockSpec((1, D), lambda i: (0, 0)),
        ],
        out_specs=pl.BlockSpec((R, D), lambda i: (i, 0)),
        out_shape=jax.ShapeDtypeStruct((N, D), jnp.float32),
    )(parts, node_feature, w1a, w1b, b1.reshape(1, H),
      W2, b2.reshape(1, H), W3, b3.reshape(1, D))


@jax.jit
def kernel(node_feature, edge_index, W1, b1, W2, b2, W3, b3):
    src = edge_index[0]
    dst = edge_index[1]
    parts = _sc_aggregate(node_feature, src, dst)
    return _mlp(parts, node_feature, W1, b1, W2, b2, W3, b3)


# speedup vs baseline: 10.6963x; 1.4064x over previous
import functools

import jax
import jax.numpy as jnp
from jax import lax
from jax.experimental import pallas as pl
from jax.experimental.pallas import tpu as pltpu
from jax.experimental.pallas import tpu_sc as plsc

N = 10000
E = 320000
D = 128
H = 128

NC = 2
NS = 16
NW = NC * NS
E_PER_TILE = E // NW
CHUNK = 125
N_CHUNKS = E_PER_TILE // CHUNK
NPASS = 2
PASS_CHUNKS = N_CHUNKS // NPASS
ROWS_MAIN = 624
ZROWS = 16


def _sc_body(nf_hbm, src_hbm, dst_hbm, out_hbm,
             acc, src_v, dst_v, rows_a, rows_b,
             gsem_a, gsem_b, ssem_a, ssem_b):
    cid = lax.axis_index("c")
    sid = lax.axis_index("s")
    wid = sid * NC + cid

    zvec = jnp.zeros((16,), jnp.float32)

    def zero_body(i, carry):
        r = i // (D // 16)
        col = (i % (D // 16)) * 16
        rows_a[r, pl.ds(col, 16)] = zvec
        return carry

    lax.fori_loop(0, 96 * (D // 16), zero_body, 0)

    row0 = sid * ROWS_MAIN

    def zcopy_body(j, carry):
        pltpu.sync_copy(rows_a.at[pl.ds(0, 96)],
                        acc.at[pl.ds(row0 + j * 96, 96)])
        return carry

    lax.fori_loop(0, 6, zcopy_body, 0)
    pltpu.sync_copy(rows_a.at[pl.ds(0, 48)],
                    acc.at[pl.ds(row0 + 576, 48)])

    @pl.when(sid == NS - 1)
    def _zero_tail():
        pltpu.sync_copy(rows_a.at[pl.ds(0, ZROWS)],
                        acc.at[pl.ds(N - ZROWS, ZROWS)])

    plsc.subcore_barrier()

    def g_start(j, buf, sem):
        pltpu.async_copy(nf_hbm.at[src_v.at[j]], buf, sem)

    def g_wait(buf, sem):
        pltpu.make_async_copy(nf_hbm.at[src_v.at[0]], buf, sem).wait()

    def s_start(j, buf, sem):
        pltpu.async_copy(buf, acc.at[dst_v.at[j]], sem, add=True)

    def s_wait(buf, sem):
        pltpu.make_async_copy(buf, acc.at[dst_v.at[0]], sem).wait()

    for p in range(NPASS):
        pltpu.sync_copy(src_hbm.at[wid, pl.ds(p * PASS_CHUNKS, PASS_CHUNKS)],
                        src_v)
        pltpu.sync_copy(dst_hbm.at[wid, pl.ds(p * PASS_CHUNKS, PASS_CHUNKS)],
                        dst_v)

        g_start(0, rows_a, gsem_a)
        g_wait(rows_a, gsem_a)
        s_start(0, rows_a, ssem_a)
        g_start(1, rows_b, gsem_b)

        def pipe_body(q, carry):
            j = 2 * q + 1
            g_wait(rows_b, gsem_b)
            s_start(j, rows_b, ssem_b)
            s_wait(rows_a, ssem_a)
            g_start(j + 1, rows_a, gsem_a)
            g_wait(rows_a, gsem_a)
            s_start(j + 1, rows_a, ssem_a)
            s_wait(rows_b, ssem_b)
            g_start(j + 2, rows_b, gsem_b)
            return carry

        lax.fori_loop(0, PASS_CHUNKS // 2 - 1, pipe_body, 0)

        g_wait(rows_b, gsem_b)
        s_start(PASS_CHUNKS - 1, rows_b, ssem_b)
        s_wait(rows_a, ssem_a)
        s_wait(rows_b, ssem_b)

    plsc.subcore_barrier()

    pltpu.sync_copy(acc.at[pl.ds(row0, ROWS_MAIN)],
                    out_hbm.at[cid, pl.ds(row0, ROWS_MAIN)])

    @pl.when(sid == NS - 1)
    def _write_tail():
        pltpu.sync_copy(acc.at[pl.ds(N - ZROWS, ZROWS)],
                        out_hbm.at[cid, pl.ds(N - ZROWS, ZROWS)])


def _sc_aggregate(node_feature, src, dst):
    mesh = plsc.VectorSubcoreMesh(core_axis_name="c", subcore_axis_name="s")
    k = pl.kernel(
        _sc_body,
        out_type=jax.ShapeDtypeStruct((NC, N, D), jnp.float32),
        mesh=mesh,
        scratch_types=[
            pltpu.VMEM_SHARED((N, D), jnp.float32),
            pltpu.VMEM((PASS_CHUNKS, CHUNK), jnp.int32),
            pltpu.VMEM((PASS_CHUNKS, CHUNK), jnp.int32),
            pltpu.VMEM((CHUNK, D), jnp.float32),
            pltpu.VMEM((CHUNK, D), jnp.float32),
            pltpu.SemaphoreType.DMA,
            pltpu.SemaphoreType.DMA,
            pltpu.SemaphoreType.DMA,
            pltpu.SemaphoreType.DMA,
        ],
    )
    return k(node_feature,
             src.reshape(NW, N_CHUNKS, CHUNK),
             dst.reshape(NW, N_CHUNKS, CHUNK))


def _mlp_body(parts_ref, nf_ref, w1a_ref, w1b_ref, b1_ref,
              w2_ref, b2_ref, w3_ref, b3_ref, out_ref):
    agg = parts_ref[0] + parts_ref[1]
    h = jnp.dot(agg, w1a_ref[...], preferred_element_type=jnp.float32)
    h += jnp.dot(nf_ref[...], w1b_ref[...], preferred_element_type=jnp.float32)
    h = jnp.maximum(h + b1_ref[...], 0.0)
    h = jnp.dot(h, w2_ref[...], preferred_element_type=jnp.float32)
    h = jnp.maximum(h + b2_ref[...], 0.0)
    h = jnp.dot(h, w3_ref[...], preferred_element_type=jnp.float32)
    out_ref[...] = h + b3_ref[...]


def _mlp(parts, node_feature, W1, b1, W2, b2, W3, b3):
    R = 1000
    grid = (N // R,)
    w1a = W1[:D]
    w1b = W1[D:]
    return pl.pallas_call(
        _mlp_body,
        grid=grid,
        in_specs=[
            pl.BlockSpec((NC, R, D), lambda i: (0, i, 0)),
            pl.BlockSpec((R, D), lambda i: (i, 0)),
            pl.BlockSpec((D, H), lambda i: (0, 0)),
            pl.BlockSpec((D, H), lambda i: (0, 0)),
            pl.BlockSpec((1, H), lambda i: (0, 0)),
            pl.BlockSpec((H, H), lambda i: (0, 0)),
            pl.BlockSpec((1, H), lambda i: (0, 0)),
            pl.BlockSpec((H, D), lambda i: (0, 0)),
            pl.BlockSpec((1, D), lambda i: (0, 0)),
        ],
        out_specs=pl.BlockSpec((R, D), lambda i: (i, 0)),
        out_shape=jax.ShapeDtypeStruct((N, D), jnp.float32),
    )(parts, node_feature, w1a, w1b, b1.reshape(1, H),
      W2, b2.reshape(1, H), W3, b3.reshape(1, D))


@jax.jit
def kernel(node_feature, edge_index, W1, b1, W2, b2, W3, b3):
    src = edge_index[0]
    dst = edge_index[1]
    parts = _sc_aggregate(node_feature, src, dst)
    return _mlp(parts, node_feature, W1, b1, W2, b2, W3, b3)
